# baseline (speedup 1.0000x reference)
import functools

import jax
import jax.numpy as jnp
from jax import lax
from jax.experimental import pallas as pl
from jax.experimental.pallas import tpu as pltpu
from jax.experimental.pallas import tpu_sc as plsc

_NC, _NS = 2, 16
_NW = _NC * _NS
_L = 16


def _proj_tc(feat, W1l, W1r):
    N, D = feat.shape
    RF = 2000
    dn = (((1,), (0,)), ((), ()))

    def body(f_ref, w_ref, p_ref):
        x2 = f_ref[...].reshape(RF // 2, 2 * D)
        p_ref[...] = lax.dot_general(x2, w_ref[...], dn,
                                     preferred_element_type=jnp.float32)

    return pl.pallas_call(
        body,
        grid=(N // RF,),
        in_specs=[pl.BlockSpec((RF, D), lambda i: (i, 0)),
                  pl.BlockSpec((2 * D, D), lambda i: (0, 0))],
        out_specs=pl.BlockSpec((RF // 2, D), lambda i: (i, 0)),
        out_shape=jax.ShapeDtypeStruct((N // 2, D), jnp.float32),
    )(feat, jnp.concatenate([W1l, W1r], axis=0))


_SUB = 80


def _enc1_all_sc(adjT2, S, P, H):
    N = adjT2.shape[0] * _SUB // S
    V = 160
    K = V // _SUB
    rows_per_slot = N // _SUB
    nchunk = N // V
    iters = -(-nchunk // _NW)
    NG = S * K + K
    mesh = plsc.VectorSubcoreMesh(core_axis_name="c", subcore_axis_name="s")

    @functools.partial(
        pl.kernel,
        out_type=(jax.ShapeDtypeStruct((N, H // 2), jnp.int32),
                  jax.ShapeDtypeStruct((N, 16), jnp.int32)),
        mesh=mesh,
        scratch_types=[
            [[pltpu.VMEM((K, _SUB), jnp.int32) for _ in range(S)]
             for _ in range(4)],
            [pltpu.VMEM((K, _SUB), jnp.int32) for _ in range(2)],
            [[pltpu.VMEM((_SUB, H), jnp.float32) for _ in range(NG)]
             for _ in range(2)],
            [pltpu.VMEM((V, H // 2), jnp.int32) for _ in range(2)],
            [pltpu.VMEM((V, 16), jnp.int32) for _ in range(2)],
            [pltpu.SemaphoreType.DMA for _ in range(4)],
            [pltpu.SemaphoreType.DMA for _ in range(2)],
            [pltpu.SemaphoreType.DMA for _ in range(2)],
            [pltpu.SemaphoreType.DMA for _ in range(2)],
        ],
        compiler_params=pltpu.CompilerParams(use_tc_tiling_on_sc=False,
                                             needs_layout_passes=False),
    )
    def k(adjT_hbm, p4_hbm, e1_hbm, adj16_hbm,
          idx_v, ipa_v, buf_v, out_v, adjr_v, sem_i, sem_g, sem_w, sem_wa):
        wid = lax.axis_index("s") * _NC + lax.axis_index("c")
        iota = lax.iota(jnp.int32, _L)
        NVS = _SUB // _L

        def chunk_of(i):
            return wid + i * _NW

        def fire_idx(i):
            q = i % 4
            c = chunk_of(i)

            @pl.when(c < nchunk)
            def _():
                for s in range(S):
                    pltpu.async_copy(
                        adjT_hbm.at[pl.ds(s * rows_per_slot + c * K, K)],
                        idx_v[q][s], sem_i[q])

        def front(i):
            p = i % 2
            q = i % 4
            c = chunk_of(i)

            @pl.when(c < nchunk)
            def _():
                base = c * V
                if i >= 2:
                    pltpu.make_async_copy(
                        adjr_v[p], adj16_hbm.at[pl.ds(0, V)],
                        sem_wa[p]).wait()
                for s in range(S):
                    pltpu.make_async_copy(
                        adjT_hbm.at[pl.ds(s * rows_per_slot + c * K, K)],
                        idx_v[q][s], sem_i[q]).wait()

                def mkpa(jj, carry):
                    kk = jj // NVS
                    off = (jj % NVS) * _L
                    ipa_v[p][kk, pl.ds(off, _L)] = (
                        2 * (base + kk * _SUB + off) + 2 * iota)
                    return carry

                lax.fori_loop(0, K * NVS, mkpa, 0)

                def asm(j2, carry):
                    rows = j2 * _L + iota
                    kk = j2 // NVS
                    off = (j2 % NVS) * _L
                    for s in range(S):
                        sl = (kk, pl.ds(off, _L))
                        g = idx_v[q][s][sl]
                        plsc.store_scatter(
                            adjr_v[p],
                            [rows, jnp.full((_L,), s, jnp.int32)], g)
                        idx_v[q][s][sl] = g * 2 + 1
                    return carry

                lax.fori_loop(0, V // _L, asm, 0)
                for s in range(S):
                    for j in range(K):
                        pltpu.async_copy(
                            p4_hbm.at[idx_v[q][s].at[j]],
                            buf_v[p][s * K + j], sem_g[p])
                for j in range(K):
                    pltpu.async_copy(
                        p4_hbm.at[ipa_v[p].at[j]],
                        buf_v[p][S * K + j], sem_g[p])

        def back(i):
            p = i % 2
            c = chunk_of(i)

            @pl.when(c < nchunk)
            def _():
                base = c * V
                for g in range(NG):
                    pltpu.make_async_copy(
                        p4_hbm.at[ipa_v[p].at[0]], buf_v[p][g],
                        sem_g[p]).wait()
                if i >= 2:
                    pltpu.make_async_copy(
                        out_v[p], e1_hbm.at[pl.ds(0, V)], sem_w[p]).wait()

                for kk in range(K):
                    def row(r, carry, kk=kk):
                        r2 = r * 2
                        for u in range(2):
                            rsub = r2 + u
                            halves = []
                            for h in range(H // _L):
                                sl = (rsub, pl.ds(h * _L, _L))
                                acc = buf_v[p][kk][sl]
                                for s in range(1, S):
                                    acc = acc + buf_v[p][s * K + kk][sl]
                                pa = buf_v[p][S * K + kk][sl]
                                halves.append(jnp.maximum(
                                    pa + acc * (1.0 / S), 0.0))
                            packed = plsc.pack(
                                halves[0], halves[1],
                                format=plsc.PackFormat.INTERLEAVED)
                            out_v[p][kk * _SUB + rsub, :] = plsc.bitcast(
                                packed, jnp.int32)
                        return carry

                    lax.fori_loop(0, _SUB // 2, row, 0)
                pltpu.async_copy(out_v[p], e1_hbm.at[pl.ds(base, V)],
                                 sem_w[p])
                pltpu.async_copy(adjr_v[p], adj16_hbm.at[pl.ds(base, V)],
                                 sem_wa[p])

        fire_idx(0)
        front(0)
        fire_idx(1)
        fire_idx(2)
        for i in range(iters):
            if i + 1 < iters:
                front(i + 1)
            if i + 3 < iters:
                fire_idx(i + 3)
            back(i)
        for i in (iters - 2, iters - 1):
            if i < 0:
                continue
            p = i % 2
            c = chunk_of(i)

            @pl.when(c < nchunk)
            def _():
                pltpu.make_async_copy(
                    out_v[p], e1_hbm.at[pl.ds(0, V)], sem_w[p]).wait()
                pltpu.make_async_copy(
                    adjr_v[p], adj16_hbm.at[pl.ds(0, V)], sem_wa[p]).wait()

    return k(adjT2, P.reshape(-1, H))


def _enc2_gather_sc(nodes2, adj16, E1, S, H):
    B = nodes2.shape[0] * _SUB
    N = E1.shape[0]
    W16 = adj16.shape[1]
    Vb = _SUB
    nchunk = B // Vb
    iters = -(-nchunk // _NW)
    NVS = _SUB // _L
    mesh = plsc.VectorSubcoreMesh(core_axis_name="c", subcore_axis_name="s")

    @functools.partial(
        pl.kernel,
        out_type=jax.ShapeDtypeStruct((B, 4 * H), jnp.float32),
        mesh=mesh,
        scratch_types=[
            [pltpu.VMEM((Vb,), jnp.int32) for _ in range(6)],
            [[pltpu.VMEM((Vb,), jnp.int32) for _ in range(S)]
             for _ in range(3)],
            [pltpu.VMEM((Vb, W16), jnp.int32) for _ in range(3)],
            [pltpu.VMEM((Vb, H // 2), jnp.int32) for _ in range(3)],
            [[pltpu.VMEM((Vb, H // 2), jnp.int32) for _ in range(S)]
             for _ in range(3)],
            [pltpu.VMEM((Vb, 4 * H), jnp.float32) for _ in range(3)],
            [pltpu.SemaphoreType.DMA for _ in range(6)],
            [pltpu.SemaphoreType.DMA for _ in range(3)],
            [pltpu.SemaphoreType.DMA for _ in range(3)],
            [pltpu.SemaphoreType.DMA for _ in range(3)],
        ],
        compiler_params=pltpu.CompilerParams(use_tc_tiling_on_sc=False,
                                             needs_layout_passes=False),
    )
    def k(nodes_hbm, adj_hbm, e1_hbm, out_hbm,
          nodes_v, idx_v, adjr_v, self_v, nbuf_v, out_v,
          sem_n, sem_a, sem_g, sem_w):
        wid = lax.axis_index("s") * _NC + lax.axis_index("c")
        iota = lax.iota(jnp.int32, _L)

        def chunk_of(i):
            return wid + i * _NW

        def fire_nodes(i):
            q = i % 6
            c = chunk_of(i)

            @pl.when(c < nchunk)
            def _():
                pltpu.async_copy(nodes_hbm.at[c], nodes_v[q], sem_n[q])

        def front(i):
            p = i % 3
            q = i % 6
            c = chunk_of(i)

            @pl.when(c < nchunk)
            def _():
                pltpu.make_async_copy(nodes_hbm.at[c], nodes_v[q],
                                      sem_n[q]).wait()
                pltpu.async_copy(adj_hbm.at[nodes_v[q]], adjr_v[p], sem_a[p])
                pltpu.async_copy(e1_hbm.at[nodes_v[q]], self_v[p], sem_g[p])

        def mid(i):
            p = i % 3
            c = chunk_of(i)

            @pl.when(c < nchunk)
            def _():
                pltpu.make_async_copy(adj_hbm.at[nodes_v[0]], adjr_v[p],
                                      sem_a[p]).wait()

                def extract(j2, carry):
                    rows = j2 * _L + iota
                    off = j2 * _L
                    for s in range(S):
                        g = plsc.load_gather(
                            adjr_v[p],
                            [rows, jnp.full((_L,), s, jnp.int32)])
                        idx_v[p][s][pl.ds(off, _L)] = g
                    return carry

                lax.fori_loop(0, NVS, extract, 0)
                for s in range(S):
                    pltpu.async_copy(e1_hbm.at[idx_v[p][s]],
                                     nbuf_v[p][s], sem_g[p])

        def back(i):
            p = i % 3
            c = chunk_of(i)

            @pl.when(c < nchunk)
            def _():
                for g in range(S + 1):
                    pltpu.make_async_copy(e1_hbm.at[nodes_v[0]],
                                          nbuf_v[p][0], sem_g[p]).wait()
                if i >= 3:
                    pltpu.make_async_copy(
                        out_v[p], out_hbm.at[pl.ds(0, Vb)], sem_w[p]).wait()

                def row(r, carry):
                    s0, s1 = plsc.unpack(
                        plsc.bitcast(self_v[p][r, :], jnp.bfloat16),
                        format=plsc.PackFormat.INTERLEAVED)
                    out_v[p][r, pl.ds(0, _L)] = s0
                    out_v[p][r, pl.ds(_L, _L)] = s1
                    a0 = a1 = None
                    for s in range(S):
                        n0, n1 = plsc.unpack(
                            plsc.bitcast(nbuf_v[p][s][r, :], jnp.bfloat16),
                            format=plsc.PackFormat.INTERLEAVED)
                        a0 = n0 if a0 is None else a0 + n0
                        a1 = n1 if a1 is None else a1 + n1
                    out_v[p][r, pl.ds(H, _L)] = a0 * (1.0 / S)
                    out_v[p][r, pl.ds(H + _L, _L)] = a1 * (1.0 / S)
                    return carry

                lax.fori_loop(0, Vb, row, 0)
                pltpu.async_copy(out_v[p], out_hbm.at[pl.ds(c * Vb, Vb)],
                                 sem_w[p])

        fire_nodes(0)
        fire_nodes(1)
        front(0)
        fire_nodes(2)
        front(1)
        mid(0)
        fire_nodes(3)
        for i in range(iters):
            if i + 2 < iters:
                front(i + 2)
            if i + 1 < iters:
                mid(i + 1)
            back(i)
            if i + 4 < iters:
                fire_nodes(i + 4)
        for i in (iters - 3, iters - 2, iters - 1):
            if i < 0:
                continue
            p = i % 3
            c = chunk_of(i)

            @pl.when(c < nchunk)
            def _():
                pltpu.make_async_copy(
                    out_v[p], out_hbm.at[pl.ds(0, Vb)], sem_w[p]).wait()

    return k(nodes2, adj16, E1)


def _head_tc(comb2z, W2t, Wct):
    B, _ = comb2z.shape
    H2 = W2t.shape[0]
    C = Wct.shape[1]
    RB = 4000
    dn = (((1,), (0,)), ((), ()))

    def body(c_ref, w2_ref, wc_ref, o_ref):
        c = c_ref[...][:, :H2]
        h = jnp.maximum(
            lax.dot_general(c, w2_ref[...], dn,
                            preferred_element_type=jnp.float32), 0.0)
        o_ref[...] = lax.dot_general(h, wc_ref[...], dn,
                                     preferred_element_type=jnp.float32)

    return pl.pallas_call(
        body,
        grid=(B // RB,),
        in_specs=[pl.BlockSpec((RB, comb2z.shape[1]), lambda i: (i, 0)),
                  pl.BlockSpec(W2t.shape, lambda i: (0, 0)),
                  pl.BlockSpec(Wct.shape, lambda i: (0, 0))],
        out_specs=pl.BlockSpec((RB, C), lambda i: (i, 0)),
        out_shape=jax.ShapeDtypeStruct((B, C), jnp.float32),
    )(comb2z, W2t, Wct)


def kernel(nodes, adj, feat, W1, W2, Wc):
    N, S = adj.shape
    D = feat.shape[1]
    H = W1.shape[0]
    adjT2 = adj.T.reshape(-1, _SUB)
    nodes2 = nodes.reshape(-1, _SUB)
    Wab = jnp.concatenate([W1[:, :D].T, W1[:, D:].T], axis=1)
    zw = jnp.zeros((D, 2 * H), jnp.float32)
    W1l = jnp.concatenate([Wab, zw], axis=1)
    W1r = jnp.concatenate([zw, Wab], axis=1)
    P = _proj_tc(feat, W1l, W1r)
    E1, adj16 = _enc1_all_sc(adjT2, S, P, H)
    comb2z = _enc2_gather_sc(nodes2, adj16, E1, S, H)
    return _head_tc(comb2z, W2.T, Wc.T)


# speedup vs baseline: 1.0510x; 1.0510x over previous
import functools

import jax
import jax.numpy as jnp
from jax import lax
from jax.experimental import pallas as pl
from jax.experimental.pallas import tpu as pltpu
from jax.experimental.pallas import tpu_sc as plsc

_NC, _NS = 2, 16
_NW = _NC * _NS
_L = 16


def _proj_tc(feat, W1l, W1r):
    N, D = feat.shape
    RF = 2000
    dn = (((1,), (0,)), ((), ()))

    def body(f_ref, w_ref, p_ref):
        x2 = f_ref[...].reshape(RF // 2, 2 * D)
        p_ref[...] = lax.dot_general(x2, w_ref[...], dn,
                                     preferred_element_type=jnp.float32)

    return pl.pallas_call(
        body,
        grid=(N // RF,),
        in_specs=[pl.BlockSpec((RF, D), lambda i: (i, 0)),
                  pl.BlockSpec((2 * D, D), lambda i: (0, 0))],
        out_specs=pl.BlockSpec((RF // 2, D), lambda i: (i, 0)),
        out_shape=jax.ShapeDtypeStruct((N // 2, D), jnp.float32),
    )(feat, jnp.concatenate([W1l, W1r], axis=0))


_SUB = 80


def _enc1_all_sc(adjT2, S, P, H):
    N = adjT2.shape[0] * _SUB // S
    V = 160
    K = V // _SUB
    rows_per_slot = N // _SUB
    nchunk = N // V
    iters = -(-nchunk // _NW)
    NG = S * K + K
    mesh = plsc.VectorSubcoreMesh(core_axis_name="c", subcore_axis_name="s")

    @functools.partial(
        pl.kernel,
        out_type=(jax.ShapeDtypeStruct((N, H), jnp.float32),
                  jax.ShapeDtypeStruct((N, 16), jnp.int32)),
        mesh=mesh,
        scratch_types=[
            [[pltpu.VMEM((K, _SUB), jnp.int32) for _ in range(S)]
             for _ in range(4)],
            [pltpu.VMEM((K, _SUB), jnp.int32) for _ in range(2)],
            [[pltpu.VMEM((_SUB, H), jnp.float32) for _ in range(NG)]
             for _ in range(2)],
            [pltpu.VMEM((V, H), jnp.float32) for _ in range(2)],
            [pltpu.VMEM((V, 16), jnp.int32) for _ in range(2)],
            [pltpu.SemaphoreType.DMA for _ in range(4)],
            [pltpu.SemaphoreType.DMA for _ in range(2)],
            [pltpu.SemaphoreType.DMA for _ in range(2)],
            [pltpu.SemaphoreType.DMA for _ in range(2)],
        ],
        compiler_params=pltpu.CompilerParams(use_tc_tiling_on_sc=False,
                                             needs_layout_passes=False),
    )
    def k(adjT_hbm, p4_hbm, e1_hbm, adj16_hbm,
          idx_v, ipa_v, buf_v, out_v, adjr_v, sem_i, sem_g, sem_w, sem_wa):
        wid = lax.axis_index("s") * _NC + lax.axis_index("c")
        iota = lax.iota(jnp.int32, _L)
        NVS = _SUB // _L

        def chunk_of(i):
            return wid + i * _NW

        def fire_idx(i):
            q = i % 4
            c = chunk_of(i)

            @pl.when(c < nchunk)
            def _():
                for s in range(S):
                    pltpu.async_copy(
                        adjT_hbm.at[pl.ds(s * rows_per_slot + c * K, K)],
                        idx_v[q][s], sem_i[q])

        def front(i):
            p = i % 2
            q = i % 4
            c = chunk_of(i)

            @pl.when(c < nchunk)
            def _():
                base = c * V
                if i >= 2:
                    pltpu.make_async_copy(
                        adjr_v[p], adj16_hbm.at[pl.ds(0, V)],
                        sem_wa[p]).wait()
                for s in range(S):
                    pltpu.make_async_copy(
                        adjT_hbm.at[pl.ds(s * rows_per_slot + c * K, K)],
                        idx_v[q][s], sem_i[q]).wait()

                def mkpa(jj, carry):
                    kk = jj // NVS
                    off = (jj % NVS) * _L
                    ipa_v[p][kk, pl.ds(off, _L)] = (
                        2 * (base + kk * _SUB + off) + 2 * iota)
                    return carry

                lax.fori_loop(0, K * NVS, mkpa, 0)

                def asm(j2, carry):
                    rows = j2 * _L + iota
                    kk = j2 // NVS
                    off = (j2 % NVS) * _L
                    for s in range(S):
                        sl = (kk, pl.ds(off, _L))
                        g = idx_v[q][s][sl]
                        plsc.store_scatter(
                            adjr_v[p],
                            [rows, jnp.full((_L,), s, jnp.int32)], g)
                        idx_v[q][s][sl] = g * 2 + 1
                    return carry

                lax.fori_loop(0, V // _L, asm, 0)
                for s in range(S):
                    for j in range(K):
                        pltpu.async_copy(
                            p4_hbm.at[idx_v[q][s].at[j]],
                            buf_v[p][s * K + j], sem_g[p])
                for j in range(K):
                    pltpu.async_copy(
                        p4_hbm.at[ipa_v[p].at[j]],
                        buf_v[p][S * K + j], sem_g[p])

        def back(i):
            p = i % 2
            c = chunk_of(i)

            @pl.when(c < nchunk)
            def _():
                base = c * V
                for g in range(NG):
                    pltpu.make_async_copy(
                        p4_hbm.at[ipa_v[p].at[0]], buf_v[p][g],
                        sem_g[p]).wait()
                if i >= 2:
                    pltpu.make_async_copy(
                        out_v[p], e1_hbm.at[pl.ds(0, V)], sem_w[p]).wait()

                for kk in range(K):
                    def row(r, carry, kk=kk):
                        r2 = r * 2
                        for u in range(2):
                            rsub = r2 + u
                            for h in range(H // _L):
                                sl = (rsub, pl.ds(h * _L, _L))
                                acc = buf_v[p][kk][sl]
                                for s in range(1, S):
                                    acc = acc + buf_v[p][s * K + kk][sl]
                                pa = buf_v[p][S * K + kk][sl]
                                out_v[p][kk * _SUB + rsub,
                                         pl.ds(h * _L, _L)] = jnp.maximum(
                                    pa + acc * (1.0 / S), 0.0)
                        return carry

                    lax.fori_loop(0, _SUB // 2, row, 0)
                pltpu.async_copy(out_v[p], e1_hbm.at[pl.ds(base, V)],
                                 sem_w[p])
                pltpu.async_copy(adjr_v[p], adj16_hbm.at[pl.ds(base, V)],
                                 sem_wa[p])

        fire_idx(0)
        front(0)
        fire_idx(1)
        fire_idx(2)
        for i in range(iters):
            if i + 1 < iters:
                front(i + 1)
            if i + 3 < iters:
                fire_idx(i + 3)
            back(i)
        for i in (iters - 2, iters - 1):
            if i < 0:
                continue
            p = i % 2
            c = chunk_of(i)

            @pl.when(c < nchunk)
            def _():
                pltpu.make_async_copy(
                    out_v[p], e1_hbm.at[pl.ds(0, V)], sem_w[p]).wait()
                pltpu.make_async_copy(
                    adjr_v[p], adj16_hbm.at[pl.ds(0, V)], sem_wa[p]).wait()

    return k(adjT2, P.reshape(-1, H))


def _enc2_gather_sc(nodes2, adj16, E1, S, H):
    B = nodes2.shape[0] * _SUB
    N = E1.shape[0]
    W16 = adj16.shape[1]
    Vb = _SUB
    nchunk = B // Vb
    iters = -(-nchunk // _NW)
    NVS = _SUB // _L
    mesh = plsc.VectorSubcoreMesh(core_axis_name="c", subcore_axis_name="s")

    @functools.partial(
        pl.kernel,
        out_type=jax.ShapeDtypeStruct((B, 4 * H), jnp.float32),
        mesh=mesh,
        scratch_types=[
            [pltpu.VMEM((Vb,), jnp.int32) for _ in range(6)],
            [[pltpu.VMEM((Vb,), jnp.int32) for _ in range(S)]
             for _ in range(3)],
            [pltpu.VMEM((Vb, W16), jnp.int32) for _ in range(3)],
            [pltpu.VMEM((Vb, H), jnp.float32) for _ in range(3)],
            [[pltpu.VMEM((Vb, H), jnp.float32) for _ in range(S)]
             for _ in range(3)],
            [pltpu.VMEM((Vb, 4 * H), jnp.float32) for _ in range(3)],
            [pltpu.SemaphoreType.DMA for _ in range(6)],
            [pltpu.SemaphoreType.DMA for _ in range(3)],
            [pltpu.SemaphoreType.DMA for _ in range(3)],
            [pltpu.SemaphoreType.DMA for _ in range(3)],
        ],
        compiler_params=pltpu.CompilerParams(use_tc_tiling_on_sc=False,
                                             needs_layout_passes=False),
    )
    def k(nodes_hbm, adj_hbm, e1_hbm, out_hbm,
          nodes_v, idx_v, adjr_v, self_v, nbuf_v, out_v,
          sem_n, sem_a, sem_g, sem_w):
        wid = lax.axis_index("s") * _NC + lax.axis_index("c")
        iota = lax.iota(jnp.int32, _L)

        def chunk_of(i):
            return wid + i * _NW

        def fire_nodes(i):
            q = i % 6
            c = chunk_of(i)

            @pl.when(c < nchunk)
            def _():
                pltpu.async_copy(nodes_hbm.at[c], nodes_v[q], sem_n[q])

        def front(i):
            p = i % 3
            q = i % 6
            c = chunk_of(i)

            @pl.when(c < nchunk)
            def _():
                pltpu.make_async_copy(nodes_hbm.at[c], nodes_v[q],
                                      sem_n[q]).wait()
                pltpu.async_copy(adj_hbm.at[nodes_v[q]], adjr_v[p], sem_a[p])
                pltpu.async_copy(e1_hbm.at[nodes_v[q]], self_v[p], sem_g[p])

        def mid(i):
            p = i % 3
            c = chunk_of(i)

            @pl.when(c < nchunk)
            def _():
                pltpu.make_async_copy(adj_hbm.at[nodes_v[0]], adjr_v[p],
                                      sem_a[p]).wait()

                def extract(j2, carry):
                    rows = j2 * _L + iota
                    off = j2 * _L
                    for s in range(S):
                        g = plsc.load_gather(
                            adjr_v[p],
                            [rows, jnp.full((_L,), s, jnp.int32)])
                        idx_v[p][s][pl.ds(off, _L)] = g
                    return carry

                lax.fori_loop(0, NVS, extract, 0)
                for s in range(S):
                    pltpu.async_copy(e1_hbm.at[idx_v[p][s]],
                                     nbuf_v[p][s], sem_g[p])

        def back(i):
            p = i % 3
            c = chunk_of(i)

            @pl.when(c < nchunk)
            def _():
                for g in range(S + 1):
                    pltpu.make_async_copy(e1_hbm.at[nodes_v[0]],
                                          nbuf_v[p][0], sem_g[p]).wait()
                if i >= 3:
                    pltpu.make_async_copy(
                        out_v[p], out_hbm.at[pl.ds(0, Vb)], sem_w[p]).wait()

                def row(r, carry):
                    for h in range(H // _L):
                        out_v[p][r, pl.ds(h * _L, _L)] = (
                            self_v[p][r, pl.ds(h * _L, _L)])
                    for h in range(H // _L):
                        acc = None
                        for s in range(S):
                            v = nbuf_v[p][s][r, pl.ds(h * _L, _L)]
                            acc = v if acc is None else acc + v
                        out_v[p][r, pl.ds(H + h * _L, _L)] = acc * (1.0 / S)
                    return carry

                lax.fori_loop(0, Vb, row, 0)
                pltpu.async_copy(out_v[p], out_hbm.at[pl.ds(c * Vb, Vb)],
                                 sem_w[p])

        fire_nodes(0)
        fire_nodes(1)
        front(0)
        fire_nodes(2)
        front(1)
        mid(0)
        fire_nodes(3)
        for i in range(iters):
            if i + 2 < iters:
                front(i + 2)
            if i + 1 < iters:
                mid(i + 1)
            back(i)
            if i + 4 < iters:
                fire_nodes(i + 4)
        for i in (iters - 3, iters - 2, iters - 1):
            if i < 0:
                continue
            p = i % 3
            c = chunk_of(i)

            @pl.when(c < nchunk)
            def _():
                pltpu.make_async_copy(
                    out_v[p], out_hbm.at[pl.ds(0, Vb)], sem_w[p]).wait()

    return k(nodes2, adj16, E1)


def _head_tc(comb2z, W2t, Wct):
    B, _ = comb2z.shape
    H2 = W2t.shape[0]
    C = Wct.shape[1]
    RB = 4000
    dn = (((1,), (0,)), ((), ()))

    def body(c_ref, w2_ref, wc_ref, o_ref):
        c = c_ref[...][:, :H2]
        h = jnp.maximum(
            lax.dot_general(c, w2_ref[...], dn,
                            preferred_element_type=jnp.float32), 0.0)
        o_ref[...] = lax.dot_general(h, wc_ref[...], dn,
                                     preferred_element_type=jnp.float32)

    return pl.pallas_call(
        body,
        grid=(B // RB,),
        in_specs=[pl.BlockSpec((RB, comb2z.shape[1]), lambda i: (i, 0)),
                  pl.BlockSpec(W2t.shape, lambda i: (0, 0)),
                  pl.BlockSpec(Wct.shape, lambda i: (0, 0))],
        out_specs=pl.BlockSpec((RB, C), lambda i: (i, 0)),
        out_shape=jax.ShapeDtypeStruct((B, C), jnp.float32),
    )(comb2z, W2t, Wct)


def kernel(nodes, adj, feat, W1, W2, Wc):
    N, S = adj.shape
    D = feat.shape[1]
    H = W1.shape[0]
    adjT2 = adj.T.reshape(-1, _SUB)
    nodes2 = nodes.reshape(-1, _SUB)
    Wab = jnp.concatenate([W1[:, :D].T, W1[:, D:].T], axis=1)
    zw = jnp.zeros((D, 2 * H), jnp.float32)
    W1l = jnp.concatenate([Wab, zw], axis=1)
    W1r = jnp.concatenate([zw, Wab], axis=1)
    P = _proj_tc(feat, W1l, W1r)
    E1, adj16 = _enc1_all_sc(adjT2, S, P, H)
    comb2z = _enc2_gather_sc(nodes2, adj16, E1, S, H)
    return _head_tc(comb2z, W2.T, Wc.T)
